# baseline (speedup 1.0000x reference)
import functools

import jax
import jax.numpy as jnp
from jax import lax
from jax.experimental import pallas as pl
from jax.experimental.pallas import tpu as pltpu
from jax.experimental.pallas import tpu_sc as plsc

NC = 2
NS = 16
NW = NC * NS
N = 64 * 1024
C = 128
T = N // NW // C
D_OBS = 256
D_ACT = 16
D_R8 = 8
D_OUT = D_OBS + D_ACT + 1

_mesh = plsc.VectorSubcoreMesh(core_axis_name="c", subcore_axis_name="s")


@functools.partial(
    pl.kernel,
    mesh=_mesh,
    compiler_params=pltpu.CompilerParams(use_tc_tiling_on_sc=False),
    out_type=jax.ShapeDtypeStruct((N, D_OUT), jnp.float32),
    scratch_types=[
        pltpu.VMEM((T, C), jnp.int32),
        pltpu.VMEM((C, D_OBS), jnp.float32),
        pltpu.VMEM((C, D_OBS), jnp.float32),
        pltpu.VMEM((C, D_OBS), jnp.float32),
        pltpu.VMEM((C, D_ACT), jnp.float32),
        pltpu.VMEM((C, D_ACT), jnp.float32),
        pltpu.VMEM((C, D_ACT), jnp.float32),
        pltpu.VMEM((C, D_R8), jnp.float32),
        pltpu.VMEM((C, D_R8), jnp.float32),
        pltpu.VMEM((C, D_R8), jnp.float32),
        pltpu.SemaphoreType.DMA,
        pltpu.SemaphoreType.DMA,
        pltpu.SemaphoreType.DMA,
        pltpu.SemaphoreType.DMA,
        pltpu.SemaphoreType.DMA,
        pltpu.SemaphoreType.DMA,
    ],
)
def _gather_all(obs_hbm, act_hbm, rew8_hbm, idx_hbm, out_hbm,
                idx_v, o0, o1, o2, a0, a1, a2, r0, r1, r2,
                sg0, sg1, sg2, sw0, sw1, sw2):
    NB = 3
    obufs = (o0, o1, o2)
    abufs = (a0, a1, a2)
    rbufs = (r0, r1, r2)
    gsem = (sg0, sg1, sg2)
    wsem = (sw0, sw1, sw2)
    wid = lax.axis_index("s") * NC + lax.axis_index("c")
    pltpu.sync_copy(idx_hbm.at[pl.ds(wid * T, T)], idx_v)

    gcp = {}
    wcp = [None] * NB

    def fire(t):
        b = t % NB
        cpo = pltpu.async_copy(obs_hbm.at[idx_v.at[t]], obufs[b], gsem[b])
        cpa = pltpu.async_copy(act_hbm.at[idx_v.at[t]], abufs[b], gsem[b])
        cpr = pltpu.async_copy(rew8_hbm.at[idx_v.at[t]], rbufs[b], gsem[b])
        gcp[t] = (cpo, cpa, cpr)

    fire(0)
    for t in range(T):
        b = t % NB
        tn = t + 1
        if tn < T:
            bn = tn % NB
            if wcp[bn] is not None:
                for w in wcp[bn]:
                    w.wait()
                wcp[bn] = None
            fire(tn)
        for cp in gcp.pop(t):
            cp.wait()
        row0 = wid * T * C + t * C
        wo = pltpu.async_copy(
            obufs[b], out_hbm.at[pl.ds(row0, C), pl.ds(0, D_OBS)], wsem[b])
        wa = pltpu.async_copy(
            abufs[b], out_hbm.at[pl.ds(row0, C), pl.ds(D_OBS, D_ACT)], wsem[b])
        wr = pltpu.async_copy(
            rbufs[b].at[:, pl.ds(0, 1)],
            out_hbm.at[pl.ds(row0, C), pl.ds(D_OBS + D_ACT, 1)], wsem[b])
        wcp[b] = (wo, wa, wr)
    for ws in wcp:
        if ws is not None:
            for w in ws:
                w.wait()


def kernel(obs, actions, rewards, batch_indices):
    obs_f = obs.reshape(N, D_OBS)
    act_f = actions.reshape(N, D_ACT)
    rew8 = jnp.pad(rewards.reshape(N, 1), ((0, 0), (0, D_R8 - 1)))
    idx = batch_indices.reshape(N // C, C)
    out = _gather_all(obs_f, act_f, rew8, idx)
    return out.reshape(64, 1024, D_OUT)


# speedup vs baseline: 1.9692x; 1.9692x over previous
import functools

import jax
import jax.numpy as jnp
from jax import lax
from jax.experimental import pallas as pl
from jax.experimental.pallas import tpu as pltpu
from jax.experimental.pallas import tpu_sc as plsc

NC = 2
NS = 16
NW = NC * NS
N = 64 * 1024
C = 128
T = N // NW // C
D_OBS = 256
D_ACT = 16
D_AR = 128
D_OUT = D_OBS + D_ACT + 1
D_PAD = 384

_mesh = plsc.VectorSubcoreMesh(core_axis_name="c", subcore_axis_name="s")


@functools.partial(
    pl.kernel,
    mesh=_mesh,
    out_type=jax.ShapeDtypeStruct((N, D_PAD), jnp.float32),
    scratch_types=[
        pltpu.VMEM((T, C), jnp.int32),
        pltpu.VMEM((C, D_OBS), jnp.float32),
        pltpu.VMEM((C, D_OBS), jnp.float32),
        pltpu.VMEM((C, D_AR), jnp.float32),
        pltpu.VMEM((C, D_AR), jnp.float32),
        pltpu.SemaphoreType.DMA,
        pltpu.SemaphoreType.DMA,
        pltpu.SemaphoreType.DMA,
        pltpu.SemaphoreType.DMA,
    ],
)
def _gather_all(obs_hbm, ar_hbm, idx_hbm, out_hbm,
                idx_v, o0, o1, a0, a1, sg0, sg1, sw0, sw1):
    NB = 2
    obufs = (o0, o1)
    abufs = (a0, a1)
    gsem = (sg0, sg1)
    wsem = (sw0, sw1)
    wid = lax.axis_index("s") * NC + lax.axis_index("c")
    pltpu.sync_copy(idx_hbm.at[pl.ds(wid * T, T)], idx_v)

    gcp = {}
    wcp = [None] * NB

    def fire(t):
        b = t % NB
        cpo = pltpu.async_copy(obs_hbm.at[idx_v.at[t]], obufs[b], gsem[b])
        cpa = pltpu.async_copy(ar_hbm.at[idx_v.at[t]], abufs[b], gsem[b])
        gcp[t] = (cpo, cpa)

    fire(0)
    for t in range(T):
        b = t % NB
        tn = t + 1
        if tn < T:
            bn = tn % NB
            if wcp[bn] is not None:
                for w in wcp[bn]:
                    w.wait()
                wcp[bn] = None
            fire(tn)
        for cp in gcp.pop(t):
            cp.wait()
        row0 = wid * T * C + t * C
        wo = pltpu.async_copy(
            obufs[b], out_hbm.at[pl.ds(row0, C), pl.ds(0, D_OBS)], wsem[b])
        wa = pltpu.async_copy(
            abufs[b], out_hbm.at[pl.ds(row0, C), pl.ds(D_OBS, D_AR)], wsem[b])
        wcp[b] = (wo, wa)
    for ws in wcp:
        if ws is not None:
            for w in ws:
                w.wait()


def kernel(obs, actions, rewards, batch_indices):
    obs_f = obs.reshape(N, D_OBS)
    ar_f = jnp.concatenate(
        [actions.reshape(N, D_ACT), rewards.reshape(N, 1),
         jnp.zeros((N, D_AR - D_ACT - 1), jnp.float32)], axis=-1)
    idx = batch_indices.reshape(N // C, C)
    out = _gather_all(obs_f, ar_f, idx)
    return out[:, :D_OUT].reshape(64, 1024, D_OUT)
